# Initial kernel scaffold; baseline (speedup 1.0000x reference)
import jax
import jax.numpy as jnp
from jax.experimental import pallas as pl


def kernel(x, g, W_self0, W_neigh0, b0, W_self1, W_neigh1, b1, W_self2, W_neigh2, b2):
    raise NotImplementedError("write your pallas kernel here")


# speedup vs baseline: 3.6299x; 3.6299x over previous
import functools

import jax
import jax.numpy as jnp
from jax import lax
from jax.experimental import pallas as pl
from jax.experimental.pallas import tpu as pltpu
from jax.experimental.pallas import tpu_sc as plsc

NC = 2
NS = 16
EB = 128


def _sc_segsum(feat, srcp, dstp, n_pad, with_deg):
    n, d = feat.shape
    e_pad = srcp.shape[0]
    n_batches = e_pad // (NC * NS * EB)
    rows_per_tile = n_pad // NS
    nchunk = rows_per_tile // EB

    mesh = plsc.VectorSubcoreMesh(core_axis_name="c", subcore_axis_name="s")

    out_type = [jax.ShapeDtypeStruct((NC, n_pad, d), jnp.float32)]
    scratch = [
        pltpu.VMEM((EB,), jnp.int32),
        pltpu.VMEM((EB,), jnp.int32),
        pltpu.VMEM((EB, d), jnp.float32),
        pltpu.VMEM_SHARED((n_pad, d), jnp.float32),
        pltpu.SemaphoreType.DMA,
    ]
    if with_deg:
        out_type.append(jax.ShapeDtypeStruct((NC, n_pad), jnp.float32))
        scratch += [
            pltpu.VMEM((EB,), jnp.float32),
            pltpu.VMEM_SHARED((n_pad,), jnp.float32),
        ]

    def body(*refs):
        if with_deg:
            (feat_h, src_h, dst_h, zrow_h, zvec_h,
             acc_out, deg_out, sidx, didx, rows, acc_sh, sem,
             ones_v, deg_sh) = refs
        else:
            (feat_h, src_h, dst_h, zrow_h,
             acc_out, sidx, didx, rows, acc_sh, sem) = refs
        c = lax.axis_index("c")
        s = lax.axis_index("s")
        rbase = s * rows_per_tile

        for k in range(nchunk):
            pltpu.sync_copy(zrow_h, acc_sh.at[pl.ds(rbase + k * EB, EB)])
        if with_deg:
            pltpu.sync_copy(zvec_h, deg_sh.at[pl.ds(rbase, rows_per_tile)])
            for j in range(EB // 16):
                ones_v[pl.ds(j * 16, 16)] = jnp.ones((16,), jnp.float32)
        plsc.subcore_barrier()

        ebase = (c * NS + s) * (n_batches * EB)

        def step(b, carry):
            off = ebase + b * EB
            pltpu.sync_copy(src_h.at[pl.ds(off, EB)], sidx)
            pltpu.sync_copy(dst_h.at[pl.ds(off, EB)], didx)
            pltpu.async_copy(feat_h.at[sidx], rows, sem).wait()
            pltpu.sync_copy(rows, acc_sh.at[didx], add=True)
            if with_deg:
                pltpu.sync_copy(ones_v, deg_sh.at[didx], add=True)
            return carry

        lax.fori_loop(0, n_batches, step, 0)
        plsc.subcore_barrier()

        for k in range(nchunk):
            r0 = rbase + k * EB
            pltpu.sync_copy(acc_sh.at[pl.ds(r0, EB)],
                            acc_out.at[c, pl.ds(r0, EB)])
        if with_deg:
            pltpu.sync_copy(deg_sh.at[pl.ds(rbase, rows_per_tile)],
                            deg_out.at[c, pl.ds(rbase, rows_per_tile)])

    zrow = jnp.zeros((EB, d), jnp.float32)
    run = pl.kernel(body, out_type=out_type, mesh=mesh, scratch_types=scratch)
    if with_deg:
        zvec = jnp.zeros((rows_per_tile,), jnp.float32)
        return run(feat, srcp, dstp, zrow, zvec)
    return run(feat, srcp, dstp, zrow)[0]


def _dot(a, b):
    return jax.lax.dot_general(a, b, (((1,), (0,)), ((), ())),
                               preferred_element_type=jnp.float32)


def _tc_layer0(x, aA, aB, dA, dB, ws, wn, b, rb):
    n, d_in = x.shape
    d_out = ws.shape[1]
    grid = (n // rb,)

    def body(x_r, aA_r, aB_r, dA_r, dB_r, ws_r, wn_r, b_r, oa_r, ob_r):
        inv = 1.0 / jnp.maximum(dA_r[...] + dB_r[...], 1.0)
        nb = (aA_r[...] + aB_r[...]) * inv
        h = _dot(x_r[...], ws_r[...]) + _dot(nb, wn_r[...]) + b_r[...]
        h = jnp.maximum(h, 0.0)
        oa_r[...] = h[:, : d_out // 2]
        ob_r[...] = h[:, d_out // 2:]

    row = lambda i: (i, 0)
    fix = lambda i: (0, 0)
    return pl.pallas_call(
        body,
        grid=grid,
        in_specs=[
            pl.BlockSpec((rb, d_in), row),
            pl.BlockSpec((rb, d_in), row),
            pl.BlockSpec((rb, d_in), row),
            pl.BlockSpec((rb, 1), row),
            pl.BlockSpec((rb, 1), row),
            pl.BlockSpec((d_in, d_out), fix),
            pl.BlockSpec((d_in, d_out), fix),
            pl.BlockSpec((1, d_out), fix),
        ],
        out_specs=[
            pl.BlockSpec((rb, d_out // 2), row),
            pl.BlockSpec((rb, d_out // 2), row),
        ],
        out_shape=[
            jax.ShapeDtypeStruct((n, d_out // 2), jnp.float32),
            jax.ShapeDtypeStruct((n, d_out // 2), jnp.float32),
        ],
    )(x, aA, aB, dA, dB, ws, wn, b)


def _tc_layer1(h1a, h1b, a0A, a0B, a1A, a1B, dA, dB,
               ws_a, ws_b, wn_a, wn_b, b, wn2, rb):
    n, dh = h1a.shape
    d_out = ws_a.shape[1]
    d2 = wn2.shape[1]
    grid = (n // rb,)

    def body(h1a_r, h1b_r, a0A_r, a0B_r, a1A_r, a1B_r, dA_r, dB_r,
             wsa_r, wsb_r, wna_r, wnb_r, b_r, wn2_r, h2_r, z2_r):
        inv = 1.0 / jnp.maximum(dA_r[...] + dB_r[...], 1.0)
        nb0 = (a0A_r[...] + a0B_r[...]) * inv
        nb1 = (a1A_r[...] + a1B_r[...]) * inv
        h = (_dot(h1a_r[...], wsa_r[...]) + _dot(h1b_r[...], wsb_r[...])
             + _dot(nb0, wna_r[...]) + _dot(nb1, wnb_r[...]) + b_r[...])
        h = jnp.maximum(h, 0.0)
        h2_r[...] = h
        z2_r[...] = _dot(h, wn2_r[...])

    row = lambda i: (i, 0)
    fix = lambda i: (0, 0)
    return pl.pallas_call(
        body,
        grid=grid,
        in_specs=[
            pl.BlockSpec((rb, dh), row),
            pl.BlockSpec((rb, dh), row),
            pl.BlockSpec((rb, dh), row),
            pl.BlockSpec((rb, dh), row),
            pl.BlockSpec((rb, dh), row),
            pl.BlockSpec((rb, dh), row),
            pl.BlockSpec((rb, 1), row),
            pl.BlockSpec((rb, 1), row),
            pl.BlockSpec((dh, d_out), fix),
            pl.BlockSpec((dh, d_out), fix),
            pl.BlockSpec((dh, d_out), fix),
            pl.BlockSpec((dh, d_out), fix),
            pl.BlockSpec((1, d_out), fix),
            pl.BlockSpec((d_out, d2), fix),
        ],
        out_specs=[
            pl.BlockSpec((rb, d_out), row),
            pl.BlockSpec((rb, d2), row),
        ],
        out_shape=[
            jax.ShapeDtypeStruct((n, d_out), jnp.float32),
            jax.ShapeDtypeStruct((n, d2), jnp.float32),
        ],
    )(h1a, h1b, a0A, a0B, a1A, a1B, dA, dB, ws_a, ws_b, wn_a, wn_b, b, wn2)


def _tc_layer2(h2, aA, aB, dA, dB, ws, b, rb):
    n, dh = h2.shape
    d_out = ws.shape[1]
    grid = (n // rb,)

    def body(h2_r, aA_r, aB_r, dA_r, dB_r, ws_r, b_r, o_r):
        inv = 1.0 / jnp.maximum(dA_r[...] + dB_r[...], 1.0)
        nb = (aA_r[...] + aB_r[...]) * inv
        o_r[...] = _dot(h2_r[...], ws_r[...]) + nb + b_r[...]

    row = lambda i: (i, 0)
    fix = lambda i: (0, 0)
    return pl.pallas_call(
        body,
        grid=grid,
        in_specs=[
            pl.BlockSpec((rb, dh), row),
            pl.BlockSpec((rb, d_out), row),
            pl.BlockSpec((rb, d_out), row),
            pl.BlockSpec((rb, 1), row),
            pl.BlockSpec((rb, 1), row),
            pl.BlockSpec((dh, d_out), fix),
            pl.BlockSpec((1, d_out), fix),
        ],
        out_specs=pl.BlockSpec((rb, d_out), row),
        out_shape=jax.ShapeDtypeStruct((n, d_out), jnp.float32),
    )(h2, aA, aB, dA, dB, ws, b)


def kernel(x, g, W_self0, W_neigh0, b0, W_self1, W_neigh1, b1,
           W_self2, W_neigh2, b2):
    n, d_in = x.shape
    n_cls = W_self2.shape[1]
    src = g[0].astype(jnp.int32)
    dst = g[1].astype(jnp.int32)
    e = src.shape[0]

    grp = NC * NS * EB
    e_pad = ((e + grp - 1) // grp) * grp
    srcp = jnp.concatenate([src, jnp.zeros((e_pad - e,), jnp.int32)])
    dstp = jnp.concatenate([dst, jnp.full((e_pad - e,), n, jnp.int32)])

    n_pad = ((n + 1 + NS * EB - 1) // (NS * EB)) * (NS * EB)
    rb = 1000

    ncp = ((n_cls + 127) // 128) * 128
    wn2p = jnp.pad(W_neigh2, ((0, 0), (0, ncp - n_cls)))
    ws2p = jnp.pad(W_self2, ((0, 0), (0, ncp - n_cls)))
    b2p = jnp.pad(b2, (0, ncp - n_cls))[None, :]

    acc0, deg = _sc_segsum(x, srcp, dstp, n_pad, with_deg=True)
    dA = deg[0, :n, None]
    dB = deg[1, :n, None]

    h1a, h1b = _tc_layer0(x, acc0[0, :n], acc0[1, :n], dA, dB,
                          W_self0, W_neigh0, b0[None, :], rb)

    acc1a = _sc_segsum(h1a, srcp, dstp, n_pad, with_deg=False)
    acc1b = _sc_segsum(h1b, srcp, dstp, n_pad, with_deg=False)

    dh = h1a.shape[1]
    h2, z2 = _tc_layer1(
        h1a, h1b,
        acc1a[0, :n], acc1a[1, :n], acc1b[0, :n], acc1b[1, :n], dA, dB,
        W_self1[:dh], W_self1[dh:], W_neigh1[:dh], W_neigh1[dh:],
        b1[None, :], wn2p, rb)

    acc2 = _sc_segsum(z2, srcp, dstp, n_pad, with_deg=False)

    out = _tc_layer2(h2, acc2[0, :n], acc2[1, :n], dA, dB, ws2p, b2p, rb)
    return out[:, :n_cls]
